# baseline (speedup 1.0000x reference)
import math

import jax
import jax.numpy as jnp
from jax import lax
from jax.experimental import pallas as pl
from jax.experimental.pallas import tpu as pltpu
from jax.experimental.pallas import tpu_sc as plsc

_VOCAB = 1000000
_DIM = 64
_PAD = 128
_B = 4096
_S = 200
_G = 128
_LANES = 16
_NBLK = _VOCAB // _PAD
_TAIL = _NBLK * _PAD
_NW = 32
_SCALE = float(math.sqrt(_DIM))


def _prep_body(tabT_hbm, stage_hbm, sbuf, cbuf, *sems):
    lsems = sems[:2]
    osems = sems[2:]
    nc = 2
    wid = lax.axis_index("s") * nc + lax.axis_index("c")

    base = _NBLK // _NW
    extra = _NBLK - base * _NW
    cnt = jnp.where(wid < extra, base + 1, base)
    first = wid * base + jnp.minimum(wid, extra)

    def l_copy(t, b):
        return pltpu.make_async_copy(
            tabT_hbm.at[:, pl.ds(t * _PAD, _PAD)], sbuf.at[b], lsems[b])

    def s_copy(t, b):
        return pltpu.make_async_copy(
            cbuf.at[b], stage_hbm.at[pl.ds(t * _PAD, _PAD)], osems[b])

    ivs = [jnp.arange(16 * j, 16 * j + 16, dtype=jnp.int32) for j in range(8)]

    def compute(b):
        @pl.loop(0, _DIM, unroll=4)
        def _d(d):
            dv = jnp.full((_LANES,), d, jnp.int32)
            for j in range(8):
                v = sbuf[b, d, pl.ds(16 * j, _LANES)]
                plsc.store_scatter(cbuf.at[b], [ivs[j], dv], v * _SCALE)

    l_copy(first, 0).start()
    l_copy(first + 1, 1).start()

    @pl.loop(0, base + 1)
    def _iter(i):
        @pl.when(i < cnt)
        def _():
            t = first + i
            for b in range(2):
                @pl.when(lax.rem(i, 2) == b)
                def _():
                    l_copy(t, b).wait()

                    @pl.when(i >= 2)
                    def _():
                        s_copy(t - 2, b).wait()

                    compute(b)
                    s_copy(t, b).start()

                    @pl.when(i + 2 < cnt)
                    def _():
                        l_copy(t + 2, b).start()

    @pl.loop(cnt - 2, cnt)
    def _drain(i):
        t = first + i
        for b in range(2):
            @pl.when(lax.rem(i, 2) == b)
            def _():
                s_copy(t, b).wait()


def _gather_body(stage_hbm, idx_hbm, tail_hbm, out_hbm,
                 idx_v, tail_v, gbuf, cbuf, *sems):
    gsems = sems[:2]
    osems = sems[2:]
    nc = 2
    wid = lax.axis_index("s") * nc + lax.axis_index("c")

    pltpu.sync_copy(idx_hbm.at[:, pl.ds(wid * _G, _G)], idx_v)
    pltpu.sync_copy(tail_hbm, tail_v)

    def g_copy(c, b):
        return pltpu.make_async_copy(
            stage_hbm.at[idx_v.at[c]], gbuf.at[b], gsems[b])

    def s_copy(c, b):
        return pltpu.make_async_copy(
            cbuf.at[b], out_hbm.at[c, :, pl.ds(wid * _G, _G)], osems[b])

    ivs = [jnp.arange(16 * l, 16 * l + 16, dtype=jnp.int32) for l in range(4)]

    def compute(c, b):
        src = gbuf.at[b]
        dst = cbuf.at[b]

        @pl.loop(0, _G, unroll=4)
        def _j(j):
            jv = jnp.full((_LANES,), j, jnp.int32)
            for l in range(4):
                v = src[j, pl.ds(16 * l, _LANES)]
                plsc.store_scatter(dst, [ivs[l], jv], v)

        masks = []
        anyhit = None
        for k in range(8):
            iv = idx_v[c, pl.ds(16 * k, _LANES)]
            m = iv >= _TAIL
            masks.append((m, jnp.maximum(iv - _TAIL, 0)))
            hit = jnp.any(m)
            anyhit = hit if anyhit is None else jnp.logical_or(anyhit, hit)

        @pl.when(anyhit)
        def _():
            for k in range(8):
                m, off = masks[k]

                @pl.when(jnp.any(m))
                def _():
                    jvk = jnp.arange(16 * k, 16 * k + 16, dtype=jnp.int32)

                    @pl.loop(0, _DIM, unroll=4)
                    def _d(d):
                        dv = jnp.full((_LANES,), d, jnp.int32)
                        v = plsc.load_gather(tail_v, [off, dv], mask=m)
                        plsc.store_scatter(dst, [dv, jvk], v, mask=m)

    g_copy(0, 0).start()
    g_copy(1, 1).start()

    @pl.loop(0, _S // 2)
    def _grp(g):
        for b in range(2):
            c = g * 2 + b
            g_copy(c, b).wait()

            @pl.when(c >= 2)
            def _():
                s_copy(c - 2, b).wait()

            compute(c, b)
            s_copy(c, b).start()

            @pl.when(c + 2 < _S)
            def _():
                g_copy(c + 2, b).start()

    for c in range(_S - 2, _S):
        s_copy(c, c % 2).wait()


def kernel(input, table):
    idxT = input.T.astype(jnp.int32)
    tabT = table.T
    tail = table[_TAIL:] * _SCALE

    mesh = plsc.VectorSubcoreMesh(core_axis_name="c", subcore_axis_name="s")
    params = pltpu.CompilerParams(
        use_tc_tiling_on_sc=True, needs_layout_passes=False)

    stage = pl.kernel(
        _prep_body,
        out_type=jax.ShapeDtypeStruct((_VOCAB, _PAD), jnp.float32),
        mesh=mesh,
        scratch_types=(
            [pltpu.VMEM((2, _DIM, _PAD), jnp.float32),
             pltpu.VMEM((2, _PAD, _PAD), jnp.float32)]
            + [pltpu.SemaphoreType.DMA] * 4
        ),
        compiler_params=params,
    )(tabT)

    out = pl.kernel(
        _gather_body,
        out_type=jax.ShapeDtypeStruct((_S, _DIM, _B), jnp.float32),
        mesh=mesh,
        scratch_types=(
            [pltpu.VMEM((_S, _G), jnp.int32),
             pltpu.VMEM((_VOCAB - _TAIL, _DIM), jnp.float32),
             pltpu.VMEM((2, _G, _PAD), jnp.float32),
             pltpu.VMEM((2, _DIM, _G), jnp.float32)]
            + [pltpu.SemaphoreType.DMA] * 4
        ),
        compiler_params=params,
    )(stage, idxT, tail)
    return jnp.transpose(out, (2, 0, 1))


# speedup vs baseline: 1.4957x; 1.4957x over previous
import math

import jax
import jax.numpy as jnp
from jax import lax
from jax.experimental import pallas as pl
from jax.experimental.pallas import tpu as pltpu
from jax.experimental.pallas import tpu_sc as plsc

_VOCAB = 1000000
_DIM = 64
_PAD = 128
_B = 4096
_S = 200
_G = 128
_LANES = 16
_NBLK = _VOCAB // _PAD
_TAIL = _NBLK * _PAD
_NW = 32
_SCALE = float(math.sqrt(_DIM))


def _prep_body(tabT_hbm, stage_hbm, sbuf, cbuf, *sems):
    lsems = sems[:2]
    osems = sems[2:]
    nc = 2
    wid = lax.axis_index("s") * nc + lax.axis_index("c")

    base = _NBLK // _NW
    extra = _NBLK - base * _NW
    cnt = jnp.where(wid < extra, base + 1, base)
    first = wid * base + jnp.minimum(wid, extra)

    def l_copy(t, b):
        return pltpu.make_async_copy(
            tabT_hbm.at[:, pl.ds(t * _PAD, _PAD)], sbuf.at[b], lsems[b])

    def s_copy(t, b):
        return pltpu.make_async_copy(
            cbuf.at[b], stage_hbm.at[pl.ds(t * _PAD, _PAD)], osems[b])

    ivs = [jnp.arange(16 * j, 16 * j + 16, dtype=jnp.int32) for j in range(8)]

    def compute(b):
        @plsc.parallel_loop(0, _DIM, unroll=4)
        def _d(d):
            dv = jnp.full((_LANES,), d, jnp.int32)
            for j in range(8):
                v = sbuf[b, d, pl.ds(16 * j, _LANES)]
                plsc.store_scatter(cbuf.at[b], [ivs[j], dv], v * _SCALE)

    l_copy(first, 0).start()
    l_copy(first + 1, 1).start()

    @pl.loop(0, base + 1)
    def _iter(i):
        @pl.when(i < cnt)
        def _():
            t = first + i
            for b in range(2):
                @pl.when(lax.rem(i, 2) == b)
                def _():
                    l_copy(t, b).wait()

                    @pl.when(i >= 2)
                    def _():
                        s_copy(t - 2, b).wait()

                    compute(b)
                    s_copy(t, b).start()

                    @pl.when(i + 2 < cnt)
                    def _():
                        l_copy(t + 2, b).start()

    @pl.loop(cnt - 2, cnt)
    def _drain(i):
        t = first + i
        for b in range(2):
            @pl.when(lax.rem(i, 2) == b)
            def _():
                s_copy(t, b).wait()


def _gather_body(stage_hbm, idx_hbm, tail_hbm, out_hbm,
                 idx_v, tail_v, gbuf, cbuf, *sems):
    gsems = sems[:2]
    osems = sems[2:]
    nc = 2
    wid = lax.axis_index("s") * nc + lax.axis_index("c")

    pltpu.sync_copy(idx_hbm.at[:, pl.ds(wid * _G, _G)], idx_v)
    pltpu.sync_copy(tail_hbm, tail_v)

    def g_copy(c, b):
        return pltpu.make_async_copy(
            stage_hbm.at[idx_v.at[c]], gbuf.at[b], gsems[b])

    def s_copy(c, b):
        return pltpu.make_async_copy(
            cbuf.at[b], out_hbm.at[c, :, pl.ds(wid * _G, _G)], osems[b])

    ivs = [jnp.arange(16 * l, 16 * l + 16, dtype=jnp.int32) for l in range(4)]

    def compute(c, b):
        src = gbuf.at[b]
        dst = cbuf.at[b]

        @plsc.parallel_loop(0, _G, unroll=4)
        def _j(j):
            jv = jnp.full((_LANES,), j, jnp.int32)
            for l in range(4):
                v = src[j, pl.ds(16 * l, _LANES)]
                plsc.store_scatter(dst, [ivs[l], jv], v)

        masks = []
        anyhit = None
        for k in range(8):
            iv = idx_v[c, pl.ds(16 * k, _LANES)]
            m = iv >= _TAIL
            masks.append((m, jnp.maximum(iv - _TAIL, 0)))
            hit = jnp.any(m)
            anyhit = hit if anyhit is None else jnp.logical_or(anyhit, hit)

        @pl.when(anyhit)
        def _():
            for k in range(8):
                m, off = masks[k]

                @pl.when(jnp.any(m))
                def _():
                    jvk = jnp.arange(16 * k, 16 * k + 16, dtype=jnp.int32)

                    @pl.loop(0, _DIM, unroll=4)
                    def _d(d):
                        dv = jnp.full((_LANES,), d, jnp.int32)
                        v = plsc.load_gather(tail_v, [off, dv], mask=m)
                        plsc.store_scatter(dst, [dv, jvk], v, mask=m)

    g_copy(0, 0).start()
    g_copy(1, 1).start()

    @pl.loop(0, _S // 2)
    def _grp(g):
        for b in range(2):
            c = g * 2 + b
            g_copy(c, b).wait()

            @pl.when(c >= 2)
            def _():
                s_copy(c - 2, b).wait()

            compute(c, b)
            s_copy(c, b).start()

            @pl.when(c + 2 < _S)
            def _():
                g_copy(c + 2, b).start()

    for c in range(_S - 2, _S):
        s_copy(c, c % 2).wait()


def kernel(input, table):
    idxT = input.T.astype(jnp.int32)
    tabT = table.T
    tail = table[_TAIL:] * _SCALE

    mesh = plsc.VectorSubcoreMesh(core_axis_name="c", subcore_axis_name="s")
    params = pltpu.CompilerParams(
        use_tc_tiling_on_sc=True, needs_layout_passes=False)

    stage = pl.kernel(
        _prep_body,
        out_type=jax.ShapeDtypeStruct((_VOCAB, _PAD), jnp.float32),
        mesh=mesh,
        scratch_types=(
            [pltpu.VMEM((2, _DIM, _PAD), jnp.float32),
             pltpu.VMEM((2, _PAD, _PAD), jnp.float32)]
            + [pltpu.SemaphoreType.DMA] * 4
        ),
        compiler_params=params,
    )(tabT)

    out = pl.kernel(
        _gather_body,
        out_type=jax.ShapeDtypeStruct((_S, _DIM, _B), jnp.float32),
        mesh=mesh,
        scratch_types=(
            [pltpu.VMEM((_S, _G), jnp.int32),
             pltpu.VMEM((_VOCAB - _TAIL, _DIM), jnp.float32),
             pltpu.VMEM((2, _G, _PAD), jnp.float32),
             pltpu.VMEM((2, _DIM, _G), jnp.float32)]
            + [pltpu.SemaphoreType.DMA] * 4
        ),
        compiler_params=params,
    )(stage, idxT, tail)
    return jnp.transpose(out, (2, 0, 1))


# speedup vs baseline: 1.8815x; 1.2579x over previous
import math

import jax
import jax.numpy as jnp
from jax import lax
from jax.experimental import pallas as pl
from jax.experimental.pallas import tpu as pltpu
from jax.experimental.pallas import tpu_sc as plsc

_VOCAB = 1000000
_DIM = 64
_PAD = 128
_B = 4096
_S = 200
_G = 128
_N = _B * _S
_CH = _N // (32 * _G)
_LANES = 16
_SCALE = float(math.sqrt(_DIM))


def _sc_body(table_hbm, idx_hbm, out_hbm, idx_v, gbuf, cbuf, *sems):
    gsems = sems[:2]
    osems = sems[2:]
    nc = 2
    wid = lax.axis_index("s") * nc + lax.axis_index("c")

    pltpu.sync_copy(idx_hbm.at[pl.ds(wid * _CH, _CH)], idx_v)
    pair_base = wid * _CH * (_G // 2)

    def g_copy(c, b):
        return pltpu.make_async_copy(
            table_hbm.at[idx_v.at[c]], gbuf.at[b], gsems[b])

    def s_copy(c, b):
        return pltpu.make_async_copy(
            cbuf.at[b],
            out_hbm.at[pl.ds(pair_base + c * (_G // 2), _G // 2)], osems[b])

    def compute(b):
        @plsc.parallel_loop(0, _G // 2, unroll=4)
        def _k(k):
            for l in range(4):
                sl = pl.ds(16 * l, _LANES)
                cbuf[b, k, sl] = gbuf[b, 2 * k, sl] * _SCALE
            for l in range(4):
                sl = pl.ds(16 * l, _LANES)
                dsl = pl.ds(64 + 16 * l, _LANES)
                cbuf[b, k, dsl] = gbuf[b, 2 * k + 1, sl] * _SCALE

    g_copy(0, 0).start()
    g_copy(1, 1).start()

    @pl.loop(0, _CH // 2)
    def _grp(g):
        for b in range(2):
            c = g * 2 + b
            g_copy(c, b).wait()

            @pl.when(c >= 2)
            def _():
                s_copy(c - 2, b).wait()

            compute(b)
            s_copy(c, b).start()

            @pl.when(c + 2 < _CH)
            def _():
                g_copy(c + 2, b).start()

    for c in range(_CH - 2, _CH):
        s_copy(c, c % 2).wait()


def kernel(input, table):
    idx2d = input.reshape(_N // _G, _G).astype(jnp.int32)
    table_p = jnp.pad(table, ((0, 0), (0, _PAD - _DIM)))

    mesh = plsc.VectorSubcoreMesh(core_axis_name="c", subcore_axis_name="s")
    out3 = pl.kernel(
        _sc_body,
        out_type=jax.ShapeDtypeStruct((_N // 2, _PAD), jnp.float32),
        mesh=mesh,
        scratch_types=(
            [pltpu.VMEM((_CH, _G), jnp.int32),
             pltpu.VMEM((2, _G, _PAD), jnp.float32),
             pltpu.VMEM((2, _G // 2, _PAD), jnp.float32)]
            + [pltpu.SemaphoreType.DMA] * 4
        ),
        compiler_params=pltpu.CompilerParams(
            use_tc_tiling_on_sc=True, needs_layout_passes=False),
    )(table_p, idx2d)
    return out3.reshape(_B, _S, _DIM)


# speedup vs baseline: 2.3479x; 1.2479x over previous
import math

import jax
import jax.numpy as jnp
from jax import lax
from jax.experimental import pallas as pl
from jax.experimental.pallas import tpu as pltpu
from jax.experimental.pallas import tpu_sc as plsc

_VOCAB = 1000000
_DIM = 64
_PAD = 128
_B = 4096
_S = 200
_G = 128
_N = _B * _S
_CH = _N // (32 * _G)
_LANES = 16
_SCALE = float(math.sqrt(_DIM))


def _sc_body(table_hbm, idx_hbm, out_hbm, idx_v, gbuf, cbuf, *sems):
    gsems = sems[:2]
    osems = sems[2:]
    nc = 2
    wid = lax.axis_index("s") * nc + lax.axis_index("c")

    pltpu.sync_copy(idx_hbm.at[pl.ds(wid * _CH, _CH)], idx_v)
    row_base = wid * _CH * _G

    def g_copy(c, b):
        return pltpu.make_async_copy(
            table_hbm.at[idx_v.at[c]], gbuf.at[b], gsems[b])

    def s_copy(c, b):
        return pltpu.make_async_copy(
            cbuf.at[b],
            out_hbm.at[pl.ds(row_base + c * _G, _G)], osems[b])

    def compute(b):
        @plsc.parallel_loop(0, _G, unroll=4)
        def _k(k):
            for l in range(4):
                sl = pl.ds(16 * l, _LANES)
                cbuf[b, k, sl] = gbuf[b, k, sl] * _SCALE

    g_copy(0, 0).start()
    g_copy(1, 1).start()

    @pl.loop(0, _CH // 2)
    def _grp(g):
        for b in range(2):
            c = g * 2 + b
            g_copy(c, b).wait()

            @pl.when(c >= 2)
            def _():
                s_copy(c - 2, b).wait()

            compute(b)
            s_copy(c, b).start()

            @pl.when(c + 2 < _CH)
            def _():
                g_copy(c + 2, b).start()

    for c in range(_CH - 2, _CH):
        s_copy(c, c % 2).wait()


def kernel(input, table):
    idx2d = input.reshape(_N // _G, _G).astype(jnp.int32)
    table_p = jnp.pad(table, ((0, 0), (0, _PAD - _DIM)))

    mesh = plsc.VectorSubcoreMesh(core_axis_name="c", subcore_axis_name="s")
    out3 = pl.kernel(
        _sc_body,
        out_type=jax.ShapeDtypeStruct((_N, _DIM), jnp.float32),
        mesh=mesh,
        scratch_types=(
            [pltpu.VMEM((_CH, _G), jnp.int32),
             pltpu.VMEM((2, _G, _PAD), jnp.float32),
             pltpu.VMEM((2, _G, _DIM), jnp.float32)]
            + [pltpu.SemaphoreType.DMA] * 4
        ),
        compiler_params=pltpu.CompilerParams(
            use_tc_tiling_on_sc=True, needs_layout_passes=False),
    )(table_p, idx2d)
    return out3.reshape(_B, _S, _DIM)
